# baseline (speedup 1.0000x reference)
import functools

import jax
import jax.numpy as jnp
from jax import lax
from jax.experimental import pallas as pl
from jax.experimental.pallas import tpu as pltpu

_HP = lax.Precision.HIGHEST
_BF = jnp.bfloat16
_DN = (((1,), (1,)), ((), ()))
_STD = (((1,), (0,)), ((), ()))


def _fused_body(n_frames, hw,
                x_ref, wA2T_ref, bA2T_ref, wB2T_ref, bB2T_ref,
                wD2T_ref, bD2T_ref, scale_ref,
                wA2C_ref, wB2C_ref, wD2C_ref,
                bA2C_ref, bB2C_ref, bD2C_ref,
                o_ref):
    N = n_frames
    f32 = jnp.float32
    NC = x_ref.shape[1] * x_ref.shape[2]
    C = x_ref.shape[2]

    Xb = x_ref[0].reshape(NC, hw)

    G = lax.dot_general(Xb, Xb, _DN, preferred_element_type=f32)

    gt = None
    s = None
    for m in range(N):
        for p_ in range(N):
            wab = sum(wA2T_ref[n, m] * wB2T_ref[n, p_] for n in range(N))
            blk = G[m * C:(m + 1) * C, p_ * C:(p_ + 1) * C]
            t = blk * wab
            gt = t if gt is None else gt + t
            if m == p_:
                s = blk if s is None else s + blk

    rs_col = [jnp.sum(Xb[m * C:(m + 1) * C, :].astype(f32), axis=1,
                      keepdims=True)
              for m in range(N)]
    rt_col = rs_col[0]
    for m in range(1, N):
        rt_col = rt_col + rs_col[m]

    ta_col = None
    tb_col = None
    for m in range(N):
        ca = sum(bB2T_ref[n] * wA2T_ref[n, m] for n in range(N))
        cb = sum(bA2T_ref[n] * wB2T_ref[n, m] for n in range(N))
        ta_col = rs_col[m] * ca if ta_col is None else ta_col + rs_col[m] * ca
        tb_col = rs_col[m] * cb if tb_col is None else tb_col + rs_col[m] * cb

    eye = (lax.broadcasted_iota(jnp.int32, (C, C), 0)
           == lax.broadcasted_iota(jnp.int32, (C, C), 1)).astype(f32)
    _C0 = (((0,), (0,)), ((), ()))
    tb_row = lax.dot_general(tb_col, eye, _C0, precision=_HP,
                             preferred_element_type=f32)

    cab = sum(bA2T_ref[n] * bB2T_ref[n] for n in range(N))
    g2t = gt + ta_col + tb_row + hw * cab

    wa2c = wA2C_ref[...]
    wb2c = wB2C_ref[...]
    ba2c_col = bA2C_ref[...]
    bb2c_row = bB2C_ref[...]
    a_s = lax.dot_general(wa2c, s, _STD, precision=_HP, preferred_element_type=f32)
    g2c = lax.dot_general(a_s, wb2c, _DN, precision=_HP, preferred_element_type=f32)
    u_col = lax.dot_general(wa2c, rt_col, _STD, precision=_HP,
                            preferred_element_type=f32)
    v_row = lax.dot_general(rt_col, wb2c, (((0,), (1,)), ((), ())),
                            precision=_HP,
                            preferred_element_type=f32)
    g2c = (g2c + u_col * bb2c_row + ba2c_col * v_row
           + (N * hw) * (ba2c_col * bb2c_row))

    inv_sc = 1.0 / scale_ref[0, 0]

    def _softmax(z):
        z = z * inv_sc
        z = z - jnp.max(z, axis=1, keepdims=True)
        e = jnp.exp(z)
        return e / jnp.sum(e, axis=1, keepdims=True)

    m2t = _softmax(g2t)
    m2c = _softmax(g2c)

    wd = lax.dot_general(m2c, wD2C_ref[...], _STD, precision=_HP,
                         preferred_element_type=f32)
    bv_col = lax.dot_general(m2c, bD2C_ref[...], _STD, precision=_HP,
                             preferred_element_type=f32)

    rows = []
    for n in range(N):
        blocks = []
        for m in range(N):
            blk = m2t * wD2T_ref[n, m]
            if n == m:
                blk = blk + wd
            blocks.append(blk)
        rows.append(jnp.concatenate(blocks, axis=1))
    bigm = jnp.concatenate(rows, axis=0).astype(_BF)

    bias_col = jnp.concatenate([bv_col + bD2T_ref[n] for n in range(N)],
                               axis=0)

    O = lax.dot_general(bigm, Xb, _STD, preferred_element_type=f32)
    o_ref[0] = ((Xb + O) + bias_col).reshape(N, C, hw).astype(o_ref.dtype)


def kernel(x, wA2T, bA2T, wA2C, bA2C, wB2T, bB2T, wB2C, bB2C,
           wD2T, bD2T, wD2C, bD2C, scale):
    B, N, C, H, W = x.shape
    HW = H * W
    f32 = jnp.float32
    x4 = x.astype(_BF).reshape(B, N, C, HW)

    smem = pl.BlockSpec(memory_space=pltpu.MemorySpace.SMEM)
    slab = pl.BlockSpec((1, N, C, HW), lambda b: (b, 0, 0, 0))
    wcc = pl.BlockSpec((C, C), lambda b: (0, 0))
    col = pl.BlockSpec((C, 1), lambda b: (0, 0))
    row = pl.BlockSpec((1, C), lambda b: (0, 0))

    cost = pl.CostEstimate(
        flops=int(B * HW * (4 * N * N * C * C + 4 * N * C)),
        transcendentals=int(2 * B * C * C),
        bytes_accessed=int(4 * 2 * B * N * C * HW))

    out = pl.pallas_call(
        functools.partial(_fused_body, N, HW),
        out_shape=jax.ShapeDtypeStruct((B, N, C, HW), _BF),
        grid=(B,),
        in_specs=[slab, smem, smem, smem, smem, smem, smem, smem,
                  wcc, wcc, wcc, col, row, col],
        out_specs=slab,
        compiler_params=pltpu.CompilerParams(
            dimension_semantics=("parallel",),
            vmem_limit_bytes=60 * 1024 * 1024),
        cost_estimate=cost,
    )(x4, wA2T, bA2T, wB2T, bB2T, wD2T, bD2T,
      jnp.asarray(scale, f32).reshape(1, 1),
      wA2C, wB2C, wD2C,
      bA2C.reshape(C, 1), bB2C.reshape(1, C), bD2C.reshape(C, 1))

    return out.reshape(B, N, C, H, W).astype(f32)


# speedup vs baseline: 1.0772x; 1.0772x over previous
import functools

import jax
import jax.numpy as jnp
from jax import lax
from jax.experimental import pallas as pl
from jax.experimental.pallas import tpu as pltpu

_HP = lax.Precision.HIGHEST
_BF = jnp.bfloat16
_DN = (((1,), (1,)), ((), ()))
_STD = (((1,), (0,)), ((), ()))


def _fused_body(n_frames, hw,
                x_ref, wA2T_ref, bA2T_ref, wB2T_ref, bB2T_ref,
                wD2T_ref, bD2T_ref, scale_ref,
                wA2C_ref, wB2C_ref, wD2C_ref,
                bA2C_ref, bB2C_ref, bD2C_ref,
                o_ref):
    N = n_frames
    f32 = jnp.float32
    NC = x_ref.shape[1] * x_ref.shape[2]
    C = x_ref.shape[2]

    X = x_ref[0].reshape(NC, hw)
    Xb = X.astype(_BF)

    G = lax.dot_general(Xb, Xb, _DN, preferred_element_type=f32)

    gt = None
    s = None
    for m in range(N):
        for p_ in range(N):
            wab = sum(wA2T_ref[n, m] * wB2T_ref[n, p_] for n in range(N))
            blk = G[m * C:(m + 1) * C, p_ * C:(p_ + 1) * C]
            t = blk * wab
            gt = t if gt is None else gt + t
            if m == p_:
                s = blk if s is None else s + blk

    rs_col = [jnp.sum(X[m * C:(m + 1) * C, :], axis=1, keepdims=True)
              for m in range(N)]
    rt_col = rs_col[0]
    for m in range(1, N):
        rt_col = rt_col + rs_col[m]

    ta_col = None
    tb_col = None
    for m in range(N):
        ca = sum(bB2T_ref[n] * wA2T_ref[n, m] for n in range(N))
        cb = sum(bA2T_ref[n] * wB2T_ref[n, m] for n in range(N))
        ta_col = rs_col[m] * ca if ta_col is None else ta_col + rs_col[m] * ca
        tb_col = rs_col[m] * cb if tb_col is None else tb_col + rs_col[m] * cb

    eye = (lax.broadcasted_iota(jnp.int32, (C, C), 0)
           == lax.broadcasted_iota(jnp.int32, (C, C), 1)).astype(f32)
    _C0 = (((0,), (0,)), ((), ()))
    tb_row = lax.dot_general(tb_col, eye, _C0, precision=_HP,
                             preferred_element_type=f32)

    cab = sum(bA2T_ref[n] * bB2T_ref[n] for n in range(N))
    g2t = gt + ta_col + tb_row + hw * cab

    wa2c = wA2C_ref[...]
    wb2c = wB2C_ref[...]
    ba2c_col = bA2C_ref[...]
    bb2c_row = bB2C_ref[...]
    a_s = lax.dot_general(wa2c, s, _STD, precision=_HP, preferred_element_type=f32)
    g2c = lax.dot_general(a_s, wb2c, _DN, precision=_HP, preferred_element_type=f32)
    u_col = lax.dot_general(wa2c, rt_col, _STD, precision=_HP,
                            preferred_element_type=f32)
    v_row = lax.dot_general(rt_col, wb2c, (((0,), (1,)), ((), ())),
                            precision=_HP,
                            preferred_element_type=f32)
    g2c = (g2c + u_col * bb2c_row + ba2c_col * v_row
           + (N * hw) * (ba2c_col * bb2c_row))

    inv_sc = 1.0 / scale_ref[0, 0]

    def _softmax(z):
        z = z * inv_sc
        z = z - jnp.max(z, axis=1, keepdims=True)
        e = jnp.exp(z)
        return e / jnp.sum(e, axis=1, keepdims=True)

    m2t = _softmax(g2t)
    m2c = _softmax(g2c)

    wd = lax.dot_general(m2c, wD2C_ref[...], _STD, precision=_HP,
                         preferred_element_type=f32)
    bv_col = lax.dot_general(m2c, bD2C_ref[...], _STD, precision=_HP,
                             preferred_element_type=f32)

    rows = []
    for n in range(N):
        blocks = []
        for m in range(N):
            blk = m2t * wD2T_ref[n, m]
            if n == m:
                blk = blk + wd
            blocks.append(blk)
        rows.append(jnp.concatenate(blocks, axis=1))
    bigm = jnp.concatenate(rows, axis=0).astype(_BF)

    bias_col = jnp.concatenate([bv_col + bD2T_ref[n] for n in range(N)],
                               axis=0)

    O = lax.dot_general(bigm, Xb, _STD, preferred_element_type=f32)
    o_ref[0] = (X + O + bias_col).reshape(N, C, hw).astype(o_ref.dtype)


def kernel(x, wA2T, bA2T, wA2C, bA2C, wB2T, bB2T, wB2C, bB2C,
           wD2T, bD2T, wD2C, bD2C, scale):
    B, N, C, H, W = x.shape
    HW = H * W
    f32 = jnp.float32
    x4 = x.reshape(B, N, C, HW)

    smem = pl.BlockSpec(memory_space=pltpu.MemorySpace.SMEM)
    slab = pl.BlockSpec((1, N, C, HW), lambda b: (b, 0, 0, 0))
    wcc = pl.BlockSpec((C, C), lambda b: (0, 0))
    col = pl.BlockSpec((C, 1), lambda b: (0, 0))
    row = pl.BlockSpec((1, C), lambda b: (0, 0))

    cost = pl.CostEstimate(
        flops=int(B * HW * (4 * N * N * C * C + 4 * N * C)),
        transcendentals=int(2 * B * C * C),
        bytes_accessed=int(4 * 2 * B * N * C * HW))

    out = pl.pallas_call(
        functools.partial(_fused_body, N, HW),
        out_shape=jax.ShapeDtypeStruct((B, N, C, HW), _BF),
        grid=(B,),
        in_specs=[slab, smem, smem, smem, smem, smem, smem, smem,
                  wcc, wcc, wcc, col, row, col],
        out_specs=slab,
        compiler_params=pltpu.CompilerParams(
            dimension_semantics=("parallel",),
            vmem_limit_bytes=60 * 1024 * 1024),
        cost_estimate=cost,
    )(x4, wA2T, bA2T, wB2T, bB2T, wD2T, bD2T,
      jnp.asarray(scale, f32).reshape(1, 1),
      wA2C, wB2C, wD2C,
      bA2C.reshape(C, 1), bB2C.reshape(1, C), bD2C.reshape(C, 1))

    return out.reshape(B, N, C, H, W).astype(f32)


# speedup vs baseline: 1.0961x; 1.0175x over previous
import functools

import jax
import jax.numpy as jnp
from jax import lax
from jax.experimental import pallas as pl
from jax.experimental.pallas import tpu as pltpu

_HP = lax.Precision.HIGHEST
_BF = jnp.bfloat16
_DN = (((1,), (1,)), ((), ()))
_STD = (((1,), (0,)), ((), ()))


def _fused_body(n_frames, hw,
                x_ref, wA2T_ref, bA2T_ref, wB2T_ref, bB2T_ref,
                wD2T_ref, bD2T_ref, scale_ref,
                wA2C_ref, wB2C_ref, wD2C_ref,
                bA2C_ref, bB2C_ref, bD2C_ref,
                o_ref):
    N = n_frames
    f32 = jnp.float32
    NC = x_ref.shape[1] * x_ref.shape[2]
    C = x_ref.shape[2]

    X = x_ref[0].reshape(NC, hw)
    Xb = X.astype(_BF)

    G = lax.dot_general(Xb, Xb, _DN, preferred_element_type=f32)

    gt = None
    s = None
    for m in range(N):
        for p_ in range(N):
            wab = sum(wA2T_ref[n, m] * wB2T_ref[n, p_] for n in range(N))
            blk = G[m * C:(m + 1) * C, p_ * C:(p_ + 1) * C]
            t = blk * wab
            gt = t if gt is None else gt + t
            if m == p_:
                s = blk if s is None else s + blk

    rs_col = [jnp.sum(X[m * C:(m + 1) * C, :], axis=1, keepdims=True)
              for m in range(N)]
    rt_col = rs_col[0]
    for m in range(1, N):
        rt_col = rt_col + rs_col[m]

    ta_col = None
    tb_col = None
    for m in range(N):
        ca = sum(bB2T_ref[n] * wA2T_ref[n, m] for n in range(N))
        cb = sum(bA2T_ref[n] * wB2T_ref[n, m] for n in range(N))
        ta_col = rs_col[m] * ca if ta_col is None else ta_col + rs_col[m] * ca
        tb_col = rs_col[m] * cb if tb_col is None else tb_col + rs_col[m] * cb

    eye = (lax.broadcasted_iota(jnp.int32, (C, C), 0)
           == lax.broadcasted_iota(jnp.int32, (C, C), 1)).astype(f32)
    _C0 = (((0,), (0,)), ((), ()))
    tb_row = lax.dot_general(tb_col, eye, _C0, precision=_HP,
                             preferred_element_type=f32)

    cab = sum(bA2T_ref[n] * bB2T_ref[n] for n in range(N))
    g2t = gt + ta_col + tb_row + hw * cab

    wa2c = wA2C_ref[...]
    wb2c = wB2C_ref[...]
    ba2c_col = bA2C_ref[...]
    bb2c_row = bB2C_ref[...]
    a_s = lax.dot_general(wa2c, s, _STD, precision=_HP, preferred_element_type=f32)
    g2c = lax.dot_general(a_s, wb2c, _DN, precision=_HP, preferred_element_type=f32)
    u_col = lax.dot_general(wa2c, rt_col, _STD, precision=_HP,
                            preferred_element_type=f32)
    v_row = lax.dot_general(rt_col, wb2c, (((0,), (1,)), ((), ())),
                            precision=_HP,
                            preferred_element_type=f32)
    g2c = (g2c + u_col * bb2c_row + ba2c_col * v_row
           + (N * hw) * (ba2c_col * bb2c_row))

    inv_sc = 1.0 / scale_ref[0, 0]

    def _softmax(z):
        z = z * inv_sc
        z = z - jnp.max(z, axis=1, keepdims=True)
        e = jnp.exp(z)
        return e / jnp.sum(e, axis=1, keepdims=True)

    m2t = _softmax(g2t)
    m2c = _softmax(g2c)

    wd = lax.dot_general(m2c, wD2C_ref[...], _STD, precision=_HP,
                         preferred_element_type=f32)
    bv_col = lax.dot_general(m2c, bD2C_ref[...], _STD, precision=_HP,
                             preferred_element_type=f32)

    wdi = wd + eye
    rows = []
    for n in range(N):
        blocks = []
        for m in range(N):
            blk = m2t * wD2T_ref[n, m]
            if n == m:
                blk = blk + wdi
            blocks.append(blk)
        rows.append(jnp.concatenate(blocks, axis=1))
    bigm = jnp.concatenate(rows, axis=0).astype(_BF)

    bias_col = jnp.concatenate([bv_col + bD2T_ref[n] for n in range(N)],
                               axis=0)

    O = lax.dot_general(bigm, Xb, _STD, preferred_element_type=f32)
    o_ref[0] = (O + bias_col).reshape(N, C, hw).astype(o_ref.dtype)


def kernel(x, wA2T, bA2T, wA2C, bA2C, wB2T, bB2T, wB2C, bB2C,
           wD2T, bD2T, wD2C, bD2C, scale):
    B, N, C, H, W = x.shape
    HW = H * W
    f32 = jnp.float32
    x4 = x.reshape(B, N, C, HW)

    smem = pl.BlockSpec(memory_space=pltpu.MemorySpace.SMEM)
    slab = pl.BlockSpec((1, N, C, HW), lambda b: (b, 0, 0, 0))
    wcc = pl.BlockSpec((C, C), lambda b: (0, 0))
    col = pl.BlockSpec((C, 1), lambda b: (0, 0))
    row = pl.BlockSpec((1, C), lambda b: (0, 0))

    cost = pl.CostEstimate(
        flops=int(B * HW * (4 * N * N * C * C + 4 * N * C)),
        transcendentals=int(2 * B * C * C),
        bytes_accessed=int(4 * 2 * B * N * C * HW))

    out = pl.pallas_call(
        functools.partial(_fused_body, N, HW),
        out_shape=jax.ShapeDtypeStruct((B, N, C, HW), _BF),
        grid=(B,),
        in_specs=[slab, smem, smem, smem, smem, smem, smem, smem,
                  wcc, wcc, wcc, col, row, col],
        out_specs=slab,
        compiler_params=pltpu.CompilerParams(
            dimension_semantics=("parallel",),
            vmem_limit_bytes=60 * 1024 * 1024),
        cost_estimate=cost,
    )(x4, wA2T, bA2T, wB2T, bB2T, wD2T, bD2T,
      jnp.asarray(scale, f32).reshape(1, 1),
      wA2C, wB2C, wD2C,
      bA2C.reshape(C, 1), bB2C.reshape(1, C), bD2C.reshape(C, 1))

    return out.reshape(B, N, C, H, W).astype(f32)
